# Initial kernel scaffold; baseline (speedup 1.0000x reference)
import jax
import jax.numpy as jnp
from jax.experimental import pallas as pl


def kernel(z_e, codebook):
    raise NotImplementedError("write your pallas kernel here")


# speedup vs baseline: 1.6157x; 1.6157x over previous
import functools

import jax
import jax.numpy as jnp
from jax import lax
from jax.experimental import pallas as pl
from jax.experimental.pallas import tpu as pltpu
from jax.experimental.pallas import tpu_sc as plsc

_NT = 256


def _half_pick(d2h):
    mn = jnp.min(d2h, axis=1, keepdims=True)
    m0 = jnp.maximum(mn, 0.0)
    s0 = jnp.sqrt(m0)
    mb = lax.bitcast_convert_type(m0, jnp.int32)
    m1 = lax.bitcast_convert_type(mb + 1, jnp.float32)
    m2 = lax.bitcast_convert_type(mb + 2, jnp.float32)
    m3 = lax.bitcast_convert_type(mb + 3, jnp.float32)
    h = jnp.where(
        jnp.sqrt(m3) == s0, m3,
        jnp.where(jnp.sqrt(m2) == s0, m2,
                  jnp.where(jnp.sqrt(m1) == s0, m1, m0)))
    k = d2h.shape[1]
    ks = lax.broadcasted_iota(jnp.int32, d2h.shape, 1).astype(jnp.float32)
    idx = jnp.min(jnp.where(d2h <= h, ks, float(k)), axis=1, keepdims=True)
    return s0, idx


def _dist_argmin_body(enc_ref, cb_ref, encsq_ref, cbsq_ref, idx_ref):
    ef = enc_ref[...].astype(jnp.float32)
    e2 = (ef + ef).astype(jnp.bfloat16)
    mm2 = lax.dot_general(e2, cb_ref[...], (((1,), (1,)), ((), ())),
                          preferred_element_type=jnp.float32)
    d2 = (encsq_ref[...] - mm2) + cbsq_ref[...]
    k = d2.shape[1]
    half = k // 2
    s0a, idxa = _half_pick(d2[:, :half])
    s0b, idxb = _half_pick(d2[:, half:])
    carry = s0a.astype(jnp.bfloat16).astype(jnp.float32)
    use_b = s0b < carry
    idx = jnp.where(use_b, idxb + float(half), idxa)
    idx_ref[...] = idx.astype(jnp.int32).reshape(1, 1, _NT)


def _tc_argmin(enc_bf, cb_bf, enc_sq, cb_sq):
    n, d3 = enc_bf.shape
    k = cb_bf.shape[0]
    g = n // _NT
    out = pl.pallas_call(
        _dist_argmin_body,
        grid=(g,),
        in_specs=[
            pl.BlockSpec((_NT, d3), lambda i: (i, 0)),
            pl.BlockSpec((k, d3), lambda i: (0, 0)),
            pl.BlockSpec((_NT, 1), lambda i: (i, 0)),
            pl.BlockSpec((1, k), lambda i: (0, 0)),
        ],
        out_specs=pl.BlockSpec((1, 1, _NT), lambda i: (i, 0, 0)),
        out_shape=jax.ShapeDtypeStruct((g, 1, _NT), jnp.int32),
    )(enc_bf, cb_bf, enc_sq, cb_sq)
    return out.reshape(n)


def _sc_gather(codebook, idx):
    n = idx.shape[0]
    k, d = codebook.shape
    nw = 32
    bpw = n // nw
    ch = 128
    nch = bpw // ch
    mesh = plsc.VectorSubcoreMesh(core_axis_name="c", subcore_axis_name="s")

    @functools.partial(
        pl.kernel,
        out_type=jax.ShapeDtypeStruct((n, d), jnp.float32),
        mesh=mesh,
        compiler_params=pltpu.CompilerParams(use_tc_tiling_on_sc=False),
        scratch_types=[
            pltpu.VMEM((ch,), jnp.int32),
            pltpu.VMEM((ch, d), jnp.float32),
            pltpu.SemaphoreType.DMA,
        ],
    )
    def gk(cb_hbm, idx_hbm, out_hbm, idx_v, rows_v, sem):
        c = lax.axis_index("c")
        s = lax.axis_index("s")
        wid = s * 2 + c
        base = wid * bpw
        for j in range(nch):
            off = base + j * ch
            pltpu.sync_copy(idx_hbm.at[pl.ds(off, ch)], idx_v)
            pltpu.async_copy(cb_hbm.at[idx_v], rows_v, sem).wait()
            pltpu.sync_copy(rows_v, out_hbm.at[pl.ds(off, ch)])

    return gk(codebook, idx)


def kernel(z_e, codebook):
    b, c, h, w = z_e.shape
    n = b * h * w
    encoded = jnp.transpose(z_e, (0, 2, 3, 1)).reshape(n, c)
    enc_sq = jnp.sum(encoded * encoded, axis=1, keepdims=True)
    cb_sq = jnp.sum(codebook * codebook, axis=1)[None, :]
    idx = _tc_argmin(encoded.astype(jnp.bfloat16), codebook.astype(jnp.bfloat16),
                     enc_sq, cb_sq)
    quant = _sc_gather(codebook, idx)
    return jnp.transpose(quant.reshape(b, h, w, c), (0, 3, 1, 2))


# speedup vs baseline: 1.6319x; 1.0100x over previous
import functools

import jax
import jax.numpy as jnp
from jax import lax
from jax.experimental import pallas as pl
from jax.experimental.pallas import tpu as pltpu
from jax.experimental.pallas import tpu_sc as plsc

_NT = 1024


def _half_pick(d2h):
    mn = jnp.min(d2h, axis=1, keepdims=True)
    m0 = jnp.maximum(mn, 0.0)
    s0 = jnp.sqrt(m0)
    mb = lax.bitcast_convert_type(m0, jnp.int32)
    m1 = lax.bitcast_convert_type(mb + 1, jnp.float32)
    m2 = lax.bitcast_convert_type(mb + 2, jnp.float32)
    m3 = lax.bitcast_convert_type(mb + 3, jnp.float32)
    h = jnp.where(
        jnp.sqrt(m3) == s0, m3,
        jnp.where(jnp.sqrt(m2) == s0, m2,
                  jnp.where(jnp.sqrt(m1) == s0, m1, m0)))
    k = d2h.shape[1]
    ks = lax.broadcasted_iota(jnp.int32, d2h.shape, 1).astype(jnp.float32)
    idx = jnp.min(jnp.where(d2h <= h, ks, float(k)), axis=1, keepdims=True)
    return s0, idx


def _dist_argmin_body(enc_ref, cb_ref, encsq_ref, cbsq_ref, idx_ref):
    ef = enc_ref[...].astype(jnp.float32)
    e2 = (ef + ef).astype(jnp.bfloat16)
    mm2 = lax.dot_general(e2, cb_ref[...], (((1,), (1,)), ((), ())),
                          preferred_element_type=jnp.float32)
    d2 = (encsq_ref[...] - mm2) + cbsq_ref[...]
    k = d2.shape[1]
    half = k // 2
    s0a, idxa = _half_pick(d2[:, :half])
    s0b, idxb = _half_pick(d2[:, half:])
    carry = s0a.astype(jnp.bfloat16).astype(jnp.float32)
    use_b = s0b < carry
    idx = jnp.where(use_b, idxb + float(half), idxa)
    idx_ref[...] = idx.astype(jnp.int32).reshape(1, 1, _NT)


def _tc_argmin(enc_bf, cb_bf, enc_sq, cb_sq):
    n, d3 = enc_bf.shape
    k = cb_bf.shape[0]
    g = n // _NT
    out = pl.pallas_call(
        _dist_argmin_body,
        grid=(g,),
        in_specs=[
            pl.BlockSpec((_NT, d3), lambda i: (i, 0)),
            pl.BlockSpec((k, d3), lambda i: (0, 0)),
            pl.BlockSpec((_NT, 1), lambda i: (i, 0)),
            pl.BlockSpec((1, k), lambda i: (0, 0)),
        ],
        out_specs=pl.BlockSpec((1, 1, _NT), lambda i: (i, 0, 0)),
        out_shape=jax.ShapeDtypeStruct((g, 1, _NT), jnp.int32),
    )(enc_bf, cb_bf, enc_sq, cb_sq)
    return out.reshape(n)


def _sc_gather(codebook, idx):
    n = idx.shape[0]
    k, d = codebook.shape
    nw = 32
    bpw = n // nw
    ch = 128
    nch = bpw // ch
    mesh = plsc.VectorSubcoreMesh(core_axis_name="c", subcore_axis_name="s")

    @functools.partial(
        pl.kernel,
        out_type=jax.ShapeDtypeStruct((n, d), jnp.float32),
        mesh=mesh,
        compiler_params=pltpu.CompilerParams(use_tc_tiling_on_sc=False),
        scratch_types=[
            pltpu.VMEM((ch,), jnp.int32),
            pltpu.VMEM((ch, d), jnp.float32),
            pltpu.SemaphoreType.DMA,
        ],
    )
    def gk(cb_hbm, idx_hbm, out_hbm, idx_v, rows_v, sem):
        c = lax.axis_index("c")
        s = lax.axis_index("s")
        wid = s * 2 + c
        base = wid * bpw
        for j in range(nch):
            off = base + j * ch
            pltpu.sync_copy(idx_hbm.at[pl.ds(off, ch)], idx_v)
            pltpu.async_copy(cb_hbm.at[idx_v], rows_v, sem).wait()
            pltpu.sync_copy(rows_v, out_hbm.at[pl.ds(off, ch)])

    return gk(codebook, idx)


def kernel(z_e, codebook):
    b, c, h, w = z_e.shape
    n = b * h * w
    encoded = jnp.transpose(z_e, (0, 2, 3, 1)).reshape(n, c)
    enc_sq = jnp.sum(encoded * encoded, axis=1, keepdims=True)
    cb_sq = jnp.sum(codebook * codebook, axis=1)[None, :]
    idx = _tc_argmin(encoded.astype(jnp.bfloat16), codebook.astype(jnp.bfloat16),
                     enc_sq, cb_sq)
    quant = _sc_gather(codebook, idx)
    return jnp.transpose(quant.reshape(b, h, w, c), (0, 3, 1, 2))


# speedup vs baseline: 1.6651x; 1.0204x over previous
import functools

import jax
import jax.numpy as jnp
from jax import lax
from jax.experimental import pallas as pl
from jax.experimental.pallas import tpu as pltpu
from jax.experimental.pallas import tpu_sc as plsc

_NT = 2048


def _half_pick(d2h):
    mn = jnp.min(d2h, axis=1, keepdims=True)
    m0 = jnp.maximum(mn, 0.0)
    s0 = jnp.sqrt(m0)
    mb = lax.bitcast_convert_type(m0, jnp.int32)
    m1 = lax.bitcast_convert_type(mb + 1, jnp.float32)
    m2 = lax.bitcast_convert_type(mb + 2, jnp.float32)
    m3 = lax.bitcast_convert_type(mb + 3, jnp.float32)
    h = jnp.where(
        jnp.sqrt(m3) == s0, m3,
        jnp.where(jnp.sqrt(m2) == s0, m2,
                  jnp.where(jnp.sqrt(m1) == s0, m1, m0)))
    k = d2h.shape[1]
    ks = lax.broadcasted_iota(jnp.int32, d2h.shape, 1).astype(jnp.float32)
    idx = jnp.min(jnp.where(d2h <= h, ks, float(k)), axis=1, keepdims=True)
    return s0, idx


def _dist_argmin_body(enc_ref, cb_ref, encsq_ref, cbsq_ref, idx_ref):
    ef = enc_ref[...].astype(jnp.float32)
    e2 = (ef + ef).astype(jnp.bfloat16)
    mm2 = lax.dot_general(e2, cb_ref[...], (((1,), (1,)), ((), ())),
                          preferred_element_type=jnp.float32)
    d2 = (encsq_ref[...] - mm2) + cbsq_ref[...]
    k = d2.shape[1]
    half = k // 2
    s0a, idxa = _half_pick(d2[:, :half])
    s0b, idxb = _half_pick(d2[:, half:])
    carry = s0a.astype(jnp.bfloat16).astype(jnp.float32)
    use_b = s0b < carry
    idx = jnp.where(use_b, idxb + float(half), idxa)
    idx_ref[...] = idx.astype(jnp.int32).reshape(1, 1, _NT)


def _tc_argmin(enc_bf, cb_bf, enc_sq, cb_sq):
    n, d3 = enc_bf.shape
    k = cb_bf.shape[0]
    g = n // _NT
    out = pl.pallas_call(
        _dist_argmin_body,
        grid=(g,),
        in_specs=[
            pl.BlockSpec((_NT, d3), lambda i: (i, 0)),
            pl.BlockSpec((k, d3), lambda i: (0, 0)),
            pl.BlockSpec((_NT, 1), lambda i: (i, 0)),
            pl.BlockSpec((1, k), lambda i: (0, 0)),
        ],
        out_specs=pl.BlockSpec((1, 1, _NT), lambda i: (i, 0, 0)),
        out_shape=jax.ShapeDtypeStruct((g, 1, _NT), jnp.int32),
    )(enc_bf, cb_bf, enc_sq, cb_sq)
    return out.reshape(n)


def _sc_gather(codebook, idx):
    n = idx.shape[0]
    k, d = codebook.shape
    nw = 32
    bpw = n // nw
    ch = 128
    nch = bpw // ch
    mesh = plsc.VectorSubcoreMesh(core_axis_name="c", subcore_axis_name="s")

    @functools.partial(
        pl.kernel,
        out_type=jax.ShapeDtypeStruct((n, d), jnp.float32),
        mesh=mesh,
        compiler_params=pltpu.CompilerParams(use_tc_tiling_on_sc=False),
        scratch_types=[
            pltpu.VMEM((ch,), jnp.int32),
            pltpu.VMEM((ch, d), jnp.float32),
            pltpu.SemaphoreType.DMA,
        ],
    )
    def gk(cb_hbm, idx_hbm, out_hbm, idx_v, rows_v, sem):
        c = lax.axis_index("c")
        s = lax.axis_index("s")
        wid = s * 2 + c
        base = wid * bpw
        for j in range(nch):
            off = base + j * ch
            pltpu.sync_copy(idx_hbm.at[pl.ds(off, ch)], idx_v)
            pltpu.async_copy(cb_hbm.at[idx_v], rows_v, sem).wait()
            pltpu.sync_copy(rows_v, out_hbm.at[pl.ds(off, ch)])

    return gk(codebook, idx)


def kernel(z_e, codebook):
    b, c, h, w = z_e.shape
    n = b * h * w
    encoded = jnp.transpose(z_e, (0, 2, 3, 1)).reshape(n, c)
    enc_sq = jnp.sum(encoded * encoded, axis=1, keepdims=True)
    cb_sq = jnp.sum(codebook * codebook, axis=1)[None, :]
    idx = _tc_argmin(encoded.astype(jnp.bfloat16), codebook.astype(jnp.bfloat16),
                     enc_sq, cb_sq)
    quant = _sc_gather(codebook, idx)
    return jnp.transpose(quant.reshape(b, h, w, c), (0, 3, 1, 2))
